# baseline (speedup 1.0000x reference)
import jax
import jax.numpy as jnp
from jax.experimental import pallas as pl
from jax.experimental.pallas import tpu as pltpu

_BM = 1000
_NBUF = 5
_NOB = 4


def _fused_heads(x_hbm, wc_ref, bc_ref, ob_hbm, oi_hbm,
                 xbuf, obuf, oibuf, sx, sob, soi):
    i = pl.program_id(0)
    n_i = pl.num_programs(0)

    def x_copy(step, slot):
        return pltpu.make_async_copy(
            x_hbm.at[pl.ds(step * _BM, _BM), :], xbuf.at[slot], sx.at[slot]
        )

    def ob_copy(step, slot):
        return pltpu.make_async_copy(
            obuf.at[slot], ob_hbm.at[pl.ds(step * _BM, _BM), :], sob.at[slot]
        )

    def oi_copy(step, slot):
        return pltpu.make_async_copy(
            oibuf.at[slot], oi_hbm.at[pl.ds(step * _BM, _BM), :], soi.at[slot]
        )

    @pl.when(i == 0)
    def _prologue():
        for k in range(_NBUF - 1):
            x_copy(k, k).start()

    nxt = i + _NBUF - 1

    @pl.when(nxt < n_i)
    def _refill():
        x_copy(nxt, jax.lax.rem(nxt, _NBUF)).start()

    slot = jax.lax.rem(i, _NBUF)
    x_copy(i, slot).wait()

    oc = jnp.dot(xbuf[slot], wc_ref[...], preferred_element_type=jnp.float32)
    oc = oc + bc_ref[...]

    oslot = jax.lax.rem(i, _NOB)

    @pl.when(i >= _NOB)
    def _drain_prev():
        ob_copy(i - _NOB, oslot).wait()
        oi_copy(i - _NOB, oslot).wait()

    obuf[oslot] = oc[:, 0:320]
    oibuf[oslot] = oc[:, 320:321]
    ob_copy(i, oslot).start()
    oi_copy(i, oslot).start()

    @pl.when(i == n_i - 1)
    def _drain_all():
        for k in range(_NOB):
            step = n_i - _NOB + k
            ob_copy(step, step % _NOB).wait()
            oi_copy(step, step % _NOB).wait()


def kernel(x, W_bbox, b_bbox, W_iou, b_iou):
    if x.ndim > 2:
        x = x.reshape(x.shape[0], -1)
    n, d = x.shape
    out_b = W_bbox.shape[1]
    wc = jnp.concatenate([W_bbox, W_iou], axis=1)
    bc = jnp.concatenate([b_bbox, b_iou]).reshape(1, out_b + 1)

    grid = (n // _BM,)
    deltas, iou = pl.pallas_call(
        _fused_heads,
        grid=grid,
        in_specs=[
            pl.BlockSpec(memory_space=pltpu.MemorySpace.HBM),
            pl.BlockSpec((d, out_b + 1), lambda i: (0, 0)),
            pl.BlockSpec((1, out_b + 1), lambda i: (0, 0)),
        ],
        out_specs=[
            pl.BlockSpec(memory_space=pltpu.MemorySpace.HBM),
            pl.BlockSpec(memory_space=pltpu.MemorySpace.HBM),
        ],
        out_shape=[
            jax.ShapeDtypeStruct((n, out_b), jnp.float32),
            jax.ShapeDtypeStruct((n, 1), jnp.float32),
        ],
        scratch_shapes=[
            pltpu.VMEM((_NBUF, _BM, d), jnp.float32),
            pltpu.VMEM((_NOB, _BM, out_b), jnp.float32),
            pltpu.VMEM((_NOB, _BM, 1), jnp.float32),
            pltpu.SemaphoreType.DMA((_NBUF,)),
            pltpu.SemaphoreType.DMA((_NOB,)),
            pltpu.SemaphoreType.DMA((_NOB,)),
        ],
        compiler_params=pltpu.CompilerParams(
            dimension_semantics=("arbitrary",),
        ),
    )(x, wc, bc)
    return (deltas, iou)
